# Initial kernel scaffold; baseline (speedup 1.0000x reference)
import jax
import jax.numpy as jnp
from jax.experimental import pallas as pl


def kernel(ped_obs, vel, nodes, edge_index, edge_attr, edge_edge_index, Wn, bn, Wa, ba, Wv, bv, We, be, Wh, bh, Wc, bc, Wgl, Wgr, att, bgat, Wg, bg, step_placeholder, Wout, bout, gF, bF, enc0_Wq, enc0_bq, enc0_Wk, enc0_bk, enc0_Wv, enc0_bv, enc0_Wo, enc0_bo, enc0_g1, enc0_b1, enc0_W1, enc0_bb1, enc0_W2, enc0_bb2, enc0_g2, enc0_b2, enc1_Wq, enc1_bq, enc1_Wk, enc1_bk, enc1_Wv, enc1_bv, enc1_Wo, enc1_bo, enc1_g1, enc1_b1, enc1_W1, enc1_bb1, enc1_W2, enc1_bb2, enc1_g2, enc1_b2, W_pos):
    raise NotImplementedError("write your pallas kernel here")


# speedup vs baseline: 1.3154x; 1.3154x over previous
import functools
import math

import jax
import jax.numpy as jnp
from jax import lax
from jax.experimental import pallas as pl
from jax.experimental.pallas import tpu as pltpu

N_PED = 10000
T = 8
PATCH = 4
STRIDE = 2
L = 3
D = 128
PRED = 12
E = 160000
M = N_PED * PATCH
H = 4
DH = D // H
S = L + PRED
EPS = 1e-6


def _feat_body(nodes_ref, ang_ref, vx_ref, vy_ref, Wn_ref, bn_ref, Wa_ref,
               ba_ref, Wv_ref, bv_ref, out_ref):
    nodes = nodes_ref[...]
    ang = ang_ref[...]
    vx = vx_ref[...]
    vy = vy_ref[...]
    f0 = nodes[:, :, None] * Wn_ref[0][None, None, :] + bn_ref[...][None, None, :]
    f1 = ang[:, :, None] * Wa_ref[0][None, None, :] + ba_ref[...][None, None, :]
    f2 = (vx[:, :, None] * Wv_ref[0][None, None, :]
          + vy[:, :, None] * Wv_ref[1][None, None, :]
          + bv_ref[...][None, None, :])
    out_ref[...] = jnp.concatenate([f0, f1, f2], axis=-1)


def _featurize(nodes, ang, vx, vy, Wn, bn, Wa, ba, Wv, bv):
    P = 2000
    grid = (N_PED // P,)
    return pl.pallas_call(
        _feat_body,
        grid=grid,
        in_specs=[
            pl.BlockSpec((P, T), lambda i: (i, 0)),
            pl.BlockSpec((P, T), lambda i: (i, 0)),
            pl.BlockSpec((P, T), lambda i: (i, 0)),
            pl.BlockSpec((P, T), lambda i: (i, 0)),
            pl.BlockSpec((1, 32), lambda i: (0, 0)),
            pl.BlockSpec((32,), lambda i: (0,)),
            pl.BlockSpec((1, 32), lambda i: (0, 0)),
            pl.BlockSpec((32,), lambda i: (0,)),
            pl.BlockSpec((2, 64), lambda i: (0, 0)),
            pl.BlockSpec((64,), lambda i: (0,)),
        ],
        out_specs=pl.BlockSpec((P, T, D), lambda i: (i, 0, 0)),
        out_shape=jax.ShapeDtypeStruct((N_PED, T, D), jnp.float32),
    )(nodes, ang, vx, vy, Wn, bn, Wa, ba, Wv, bv)


def _proj_body(x_ref, Wgl_ref, Wgr_ref, hl_ref, hr_ref):
    x = x_ref[...]
    hl_ref[...] = jnp.dot(x, Wgl_ref[...], preferred_element_type=jnp.float32)
    hr_ref[...] = jnp.dot(x, Wgr_ref[...], preferred_element_type=jnp.float32)


def _proj(z1_flat, Wgl, Wgr):
    R = z1_flat.shape[0]
    P = 6000
    grid = (R // P,)
    return pl.pallas_call(
        _proj_body,
        grid=grid,
        in_specs=[
            pl.BlockSpec((P, D), lambda i: (i, 0)),
            pl.BlockSpec((D, D), lambda i: (0, 0)),
            pl.BlockSpec((D, D), lambda i: (0, 0)),
        ],
        out_specs=[
            pl.BlockSpec((P, D), lambda i: (i, 0)),
            pl.BlockSpec((P, D), lambda i: (i, 0)),
        ],
        out_shape=[
            jax.ShapeDtypeStruct((R, D), jnp.float32),
            jax.ShapeDtypeStruct((R, D), jnp.float32),
        ],
    )(z1_flat, Wgl, Wgr)


def _ln_rows(x, g, b):
    m = jnp.mean(x, axis=-1, keepdims=True)
    v = jnp.mean((x - m) ** 2, axis=-1, keepdims=True)
    return (x - m) * jax.lax.rsqrt(v + 1e-5) * g + b


def _enc_layer(xx, P, pr):
    q = jnp.dot(xx, pr['Wq'], preferred_element_type=jnp.float32) + pr['bq']
    k = jnp.dot(xx, pr['Wk'], preferred_element_type=jnp.float32) + pr['bk']
    v = jnp.dot(xx, pr['Wv'], preferred_element_type=jnp.float32) + pr['bv']
    q3 = q.reshape(P, S, D)
    k3 = k.reshape(P, S, D)
    v3 = v.reshape(P, S, D)
    scale = 1.0 / math.sqrt(DH)
    o_heads = []
    for h in range(H):
        qh = q3[:, :, h * DH:(h + 1) * DH]
        kh = k3[:, :, h * DH:(h + 1) * DH]
        vh = v3[:, :, h * DH:(h + 1) * DH]
        sc = jax.lax.dot_general(
            qh, kh, (((2,), (2,)), ((0,), (0,))),
            preferred_element_type=jnp.float32) * scale
        mx = jnp.max(sc, axis=-1, keepdims=True)
        ex = jnp.exp(sc - mx)
        aw = ex / jnp.sum(ex, axis=-1, keepdims=True)
        oh = jax.lax.dot_general(
            aw, vh, (((2,), (1,)), ((0,), (0,))),
            preferred_element_type=jnp.float32)
        o_heads.append(oh)
    o = jnp.concatenate(o_heads, axis=-1).reshape(P * S, D)
    o = jnp.dot(o, pr['Wo'], preferred_element_type=jnp.float32) + pr['bo']
    xx = xx + o
    xx = _ln_rows(xx, pr['g1'], pr['b1'])
    y = jnp.dot(xx, pr['W1'], preferred_element_type=jnp.float32) + pr['bb1']
    y = jax.nn.gelu(y)
    y = jnp.dot(y, pr['W2'], preferred_element_type=jnp.float32) + pr['bb2']
    xx = _ln_rows(xx + y, pr['g2'], pr['b2'])
    return xx


def _enc_body(gt_ref, sp_ref, wpos_ref, *rest):
    names = ['Wq', 'bq', 'Wk', 'bk', 'Wv', 'bv', 'Wo', 'bo',
             'g1', 'b1', 'W1', 'bb1', 'W2', 'bb2', 'g2', 'b2']
    p0 = {n: rest[i][...] for i, n in enumerate(names)}
    p1 = {n: rest[16 + i][...] for i, n in enumerate(names)}
    gF = rest[32][...]
    bF = rest[33][...]
    Wout = rest[34][...]
    bout = rest[35][...]
    out_ref = rest[36]

    gt = gt_ref[...]
    sp = sp_ref[...]
    P = gt.shape[0]
    ph = jnp.broadcast_to(sp[:, None, :], (P, PRED, D))
    xx = jnp.concatenate([gt, ph], axis=1) + wpos_ref[...][None, :, :]
    xx = xx.reshape(P * S, D)
    xx = _enc_layer(xx, P, p0)
    xx = _enc_layer(xx, P, p1)
    xx = _ln_rows(xx, gF, bF)
    out = jnp.dot(xx, Wout, preferred_element_type=jnp.float32) + bout
    out_ref[...] = out.reshape(P, S, 5)


def _encoder(gt, sp, W_pos, enc_params, gF, bF, Wout, bout):
    P = 400
    grid = (N_PED // P,)
    in_specs = [
        pl.BlockSpec((P, L, D), lambda i: (i, 0, 0)),
        pl.BlockSpec((P, D), lambda i: (i, 0)),
        pl.BlockSpec((S, D), lambda i: (0, 0)),
    ]
    args = [gt, sp, W_pos]
    for arr in enc_params:
        in_specs.append(pl.BlockSpec(arr.shape, lambda i, nd=arr.ndim: (0,) * nd))
        args.append(arr)
    for arr in (gF, bF, Wout, bout):
        in_specs.append(pl.BlockSpec(arr.shape, lambda i, nd=arr.ndim: (0,) * nd))
        args.append(arr)
    return pl.pallas_call(
        _enc_body,
        grid=grid,
        in_specs=in_specs,
        out_specs=pl.BlockSpec((P, S, 5), lambda i: (i, 0, 0)),
        out_shape=jax.ShapeDtypeStruct((N_PED, S, 5), jnp.float32),
    )(*args)


def _gat_layer(x, hl, hr, src, dst, ea, att, bgat, c, dvec):
    e = hl[src] + hr[dst]
    e = jnp.where(e > 0, e, 0.2 * e)
    logits = e @ att
    a = jnp.exp(logits)
    s = jax.ops.segment_sum(a, dst, num_segments=M)
    alpha = a / (s[dst] + 1e-16)
    node_emb = jax.ops.segment_sum(hl[src] * alpha[:, None], dst,
                                   num_segments=M) + bgat
    en = ea[:, None] * c[None, :] + dvec[None, :]
    agg = (jax.ops.segment_sum(node_emb[src] * en[:, 0:1], dst, num_segments=M)
           + jax.ops.segment_sum(node_emb[dst] * en[:, 1:2], src, num_segments=M))
    return agg


def _gt_body(aggm_ref, xm_ref, Wg_ref, bg_ref, out_ref):
    aggm = aggm_ref[...]
    xm = xm_ref[...]
    P = aggm.shape[0]
    a2 = aggm.reshape(P * L, D)
    g = jnp.dot(a2, Wg_ref[...], preferred_element_type=jnp.float32) + bg_ref[...]
    out_ref[...] = g.reshape(P, L, D) + xm


def _gt_kernel(aggm, xm, Wg, bg):
    P = 2000
    grid = (N_PED // P,)
    return pl.pallas_call(
        _gt_body,
        grid=grid,
        in_specs=[
            pl.BlockSpec((P, L, D), lambda i: (i, 0, 0)),
            pl.BlockSpec((P, L, D), lambda i: (i, 0, 0)),
            pl.BlockSpec((D, D), lambda i: (0, 0)),
            pl.BlockSpec((D,), lambda i: (0,)),
        ],
        out_specs=pl.BlockSpec((P, L, D), lambda i: (i, 0, 0)),
        out_shape=jax.ShapeDtypeStruct((N_PED, L, D), jnp.float32),
    )(aggm, xm, Wg, bg)


def kernel(ped_obs, vel, nodes, edge_index, edge_attr, edge_edge_index, Wn, bn,
           Wa, ba, Wv, bv, We, be, Wh, bh, Wc, bc, Wgl, Wgr, att, bgat, Wg, bg,
           step_placeholder, Wout, bout, gF, bF,
           enc0_Wq, enc0_bq, enc0_Wk, enc0_bk, enc0_Wv, enc0_bv, enc0_Wo,
           enc0_bo, enc0_g1, enc0_b1, enc0_W1, enc0_bb1, enc0_W2, enc0_bb2,
           enc0_g2, enc0_b2,
           enc1_Wq, enc1_bq, enc1_Wk, enc1_bk, enc1_Wv, enc1_bv, enc1_Wo,
           enc1_bo, enc1_g1, enc1_b1, enc1_W1, enc1_bb1, enc1_W2, enc1_bb2,
           enc1_g2, enc1_b2,
           W_pos):
    vx = vel[:, :, 0]
    vy = vel[:, :, 1]
    vpx = jnp.concatenate([vx[:, :1], vx[:, :-1]], axis=1)
    vpy = jnp.concatenate([vy[:, :1], vy[:, :-1]], axis=1)
    nrm = jnp.sqrt(vx * vx + vy * vy)
    nrm_p = jnp.sqrt(vpx * vpx + vpy * vpy)
    cos = (vpx * vx + vpy * vy) / ((nrm_p + EPS) * (nrm + EPS))
    ang = jnp.arccos(jnp.clip(cos, -1.0, 1.0))
    nf = _featurize(nodes, ang, vx, vy, Wn, bn, Wa, ba, Wv, bv)

    z1 = jnp.stack([nf[:, i * STRIDE:i * STRIDE + PATCH] for i in range(L)],
                   axis=0).reshape(L, M, D)
    hl_all, hr_all = _proj(z1.reshape(L * M, D), Wgl, Wgr)
    hl_all = hl_all.reshape(L, M, D)
    hr_all = hr_all.reshape(L, M, D)

    c = (We @ Wh @ Wc)[0]
    dvec = (be @ Wh + bh) @ Wc + bc

    aggs = []
    for i in range(L):
        ea = edge_attr[i]
        ea = jnp.where(jnp.isnan(ea) | jnp.isinf(ea), 0.0, ea)
        agg = _gat_layer(z1[i], hl_all[i], hr_all[i], edge_index[i, 0],
                         edge_index[i, 1], ea, att, bgat, c, dvec)
        aggs.append(agg)
    agg_all = jnp.stack(aggs, 0)

    aggm = jnp.transpose(agg_all.reshape(L, N_PED, PATCH, D).mean(axis=2),
                         (1, 0, 2))
    xm = jnp.transpose(z1.reshape(L, N_PED, PATCH, D).mean(axis=2), (1, 0, 2))
    gt = _gt_kernel(aggm, xm, Wg, bg)

    sp = step_placeholder[0]
    enc_params = [enc0_Wq, enc0_bq, enc0_Wk, enc0_bk, enc0_Wv, enc0_bv,
                  enc0_Wo, enc0_bo, enc0_g1, enc0_b1, enc0_W1, enc0_bb1,
                  enc0_W2, enc0_bb2, enc0_g2, enc0_b2,
                  enc1_Wq, enc1_bq, enc1_Wk, enc1_bk, enc1_Wv, enc1_bv,
                  enc1_Wo, enc1_bo, enc1_g1, enc1_b1, enc1_W1, enc1_bb1,
                  enc1_W2, enc1_bb2, enc1_g2, enc1_b2]
    return _encoder(gt, sp, W_pos, enc_params, gF, bF, Wout, bout)


# speedup vs baseline: 1.5558x; 1.1828x over previous
import functools
import math

import jax
import jax.numpy as jnp
from jax import lax
from jax.experimental import pallas as pl
from jax.experimental.pallas import tpu as pltpu
from jax.experimental.pallas import tpu_sc as plsc

N_PED = 10000
T = 8
PATCH = 4
STRIDE = 2
L = 3
D = 128
PRED = 12
E = 160000
M = N_PED * PATCH
H = 4
DH = D // H
S = L + PRED
EPS = 1e-6


def _feat_body(nodes_ref, ang_ref, vx_ref, vy_ref, Wn_ref, bn_ref, Wa_ref,
               ba_ref, Wv_ref, bv_ref, out_ref):
    nodes = nodes_ref[...]
    ang = ang_ref[...]
    vx = vx_ref[...]
    vy = vy_ref[...]
    f0 = nodes[:, :, None] * Wn_ref[0][None, None, :] + bn_ref[...][None, None, :]
    f1 = ang[:, :, None] * Wa_ref[0][None, None, :] + ba_ref[...][None, None, :]
    f2 = (vx[:, :, None] * Wv_ref[0][None, None, :]
          + vy[:, :, None] * Wv_ref[1][None, None, :]
          + bv_ref[...][None, None, :])
    out_ref[...] = jnp.concatenate([f0, f1, f2], axis=-1)


def _featurize(nodes, ang, vx, vy, Wn, bn, Wa, ba, Wv, bv):
    P = 2000
    grid = (N_PED // P,)
    return pl.pallas_call(
        _feat_body,
        grid=grid,
        in_specs=[
            pl.BlockSpec((P, T), lambda i: (i, 0)),
            pl.BlockSpec((P, T), lambda i: (i, 0)),
            pl.BlockSpec((P, T), lambda i: (i, 0)),
            pl.BlockSpec((P, T), lambda i: (i, 0)),
            pl.BlockSpec((1, 32), lambda i: (0, 0)),
            pl.BlockSpec((32,), lambda i: (0,)),
            pl.BlockSpec((1, 32), lambda i: (0, 0)),
            pl.BlockSpec((32,), lambda i: (0,)),
            pl.BlockSpec((2, 64), lambda i: (0, 0)),
            pl.BlockSpec((64,), lambda i: (0,)),
        ],
        out_specs=pl.BlockSpec((P, T, D), lambda i: (i, 0, 0)),
        out_shape=jax.ShapeDtypeStruct((N_PED, T, D), jnp.float32),
    )(nodes, ang, vx, vy, Wn, bn, Wa, ba, Wv, bv)


def _proj_body(x_ref, Wgl_ref, Wgr_ref, hl_ref, hr_ref):
    x = x_ref[...]
    hl_ref[...] = jnp.dot(x, Wgl_ref[...], preferred_element_type=jnp.float32)
    hr_ref[...] = jnp.dot(x, Wgr_ref[...], preferred_element_type=jnp.float32)


def _proj(z1_flat, Wgl, Wgr):
    R = z1_flat.shape[0]
    P = 6000
    grid = (R // P,)
    return pl.pallas_call(
        _proj_body,
        grid=grid,
        in_specs=[
            pl.BlockSpec((P, D), lambda i: (i, 0)),
            pl.BlockSpec((D, D), lambda i: (0, 0)),
            pl.BlockSpec((D, D), lambda i: (0, 0)),
        ],
        out_specs=[
            pl.BlockSpec((P, D), lambda i: (i, 0)),
            pl.BlockSpec((P, D), lambda i: (i, 0)),
        ],
        out_shape=[
            jax.ShapeDtypeStruct((R, D), jnp.float32),
            jax.ShapeDtypeStruct((R, D), jnp.float32),
        ],
    )(z1_flat, Wgl, Wgr)


def _ln_rows(x, g, b):
    m = jnp.mean(x, axis=-1, keepdims=True)
    v = jnp.mean((x - m) ** 2, axis=-1, keepdims=True)
    return (x - m) * jax.lax.rsqrt(v + 1e-5) * g + b


def _enc_layer(xx, P, pr):
    q = jnp.dot(xx, pr['Wq'], preferred_element_type=jnp.float32) + pr['bq']
    k = jnp.dot(xx, pr['Wk'], preferred_element_type=jnp.float32) + pr['bk']
    v = jnp.dot(xx, pr['Wv'], preferred_element_type=jnp.float32) + pr['bv']
    q3 = q.reshape(P, S, D)
    k3 = k.reshape(P, S, D)
    v3 = v.reshape(P, S, D)
    scale = 1.0 / math.sqrt(DH)
    o_heads = []
    for h in range(H):
        qh = q3[:, :, h * DH:(h + 1) * DH]
        kh = k3[:, :, h * DH:(h + 1) * DH]
        vh = v3[:, :, h * DH:(h + 1) * DH]
        sc = jax.lax.dot_general(
            qh, kh, (((2,), (2,)), ((0,), (0,))),
            preferred_element_type=jnp.float32) * scale
        mx = jnp.max(sc, axis=-1, keepdims=True)
        ex = jnp.exp(sc - mx)
        aw = ex / jnp.sum(ex, axis=-1, keepdims=True)
        oh = jax.lax.dot_general(
            aw, vh, (((2,), (1,)), ((0,), (0,))),
            preferred_element_type=jnp.float32)
        o_heads.append(oh)
    o = jnp.concatenate(o_heads, axis=-1).reshape(P * S, D)
    o = jnp.dot(o, pr['Wo'], preferred_element_type=jnp.float32) + pr['bo']
    xx = xx + o
    xx = _ln_rows(xx, pr['g1'], pr['b1'])
    y = jnp.dot(xx, pr['W1'], preferred_element_type=jnp.float32) + pr['bb1']
    y = jax.nn.gelu(y)
    y = jnp.dot(y, pr['W2'], preferred_element_type=jnp.float32) + pr['bb2']
    xx = _ln_rows(xx + y, pr['g2'], pr['b2'])
    return xx


def _enc_body(gt_ref, sp_ref, wpos_ref, *rest):
    names = ['Wq', 'bq', 'Wk', 'bk', 'Wv', 'bv', 'Wo', 'bo',
             'g1', 'b1', 'W1', 'bb1', 'W2', 'bb2', 'g2', 'b2']
    p0 = {n: rest[i][...] for i, n in enumerate(names)}
    p1 = {n: rest[16 + i][...] for i, n in enumerate(names)}
    gF = rest[32][...]
    bF = rest[33][...]
    Wout = rest[34][...]
    bout = rest[35][...]
    out_ref = rest[36]

    gt = gt_ref[...]
    sp = sp_ref[...]
    P = gt.shape[0]
    ph = jnp.broadcast_to(sp[:, None, :], (P, PRED, D))
    xx = jnp.concatenate([gt, ph], axis=1) + wpos_ref[...][None, :, :]
    xx = xx.reshape(P * S, D)
    xx = _enc_layer(xx, P, p0)
    xx = _enc_layer(xx, P, p1)
    xx = _ln_rows(xx, gF, bF)
    out = jnp.dot(xx, Wout, preferred_element_type=jnp.float32) + bout
    out_ref[...] = out.reshape(P, S, 5)


def _encoder(gt, sp, W_pos, enc_params, gF, bF, Wout, bout):
    P = 400
    grid = (N_PED // P,)
    in_specs = [
        pl.BlockSpec((P, L, D), lambda i: (i, 0, 0)),
        pl.BlockSpec((P, D), lambda i: (i, 0)),
        pl.BlockSpec((S, D), lambda i: (0, 0)),
    ]
    args = [gt, sp, W_pos]
    for arr in enc_params:
        in_specs.append(pl.BlockSpec(arr.shape, lambda i, nd=arr.ndim: (0,) * nd))
        args.append(arr)
    for arr in (gF, bF, Wout, bout):
        in_specs.append(pl.BlockSpec(arr.shape, lambda i, nd=arr.ndim: (0,) * nd))
        args.append(arr)
    return pl.pallas_call(
        _enc_body,
        grid=grid,
        in_specs=in_specs,
        out_specs=pl.BlockSpec((P, S, 5), lambda i: (i, 0, 0)),
        out_shape=jax.ShapeDtypeStruct((N_PED, S, 5), jnp.float32),
    )(*args)


E_PAD = 163840
NC = 2
NS = 16
NW = NC * NS
G_CH = 40
S_CH = 80
S_SZ = 40960
SP_DATA = 10000
SP_ROWS = SP_DATA + NS
SC2_CH = 160
SC2_W = 64
_SC_MESH = dict(core_axis_name="c", subcore_axis_name="s")


def _sc_gather(tabA, tabB, idx):
    NT = 6
    idx4 = idx.reshape(NT, NW, G_CH, 128)

    @functools.partial(
        pl.kernel,
        out_type=jax.ShapeDtypeStruct((NT, NW, G_CH, 128, 128), jnp.float32),
        mesh=plsc.VectorSubcoreMesh(**_SC_MESH),
        scratch_types=[
            pltpu.VMEM((G_CH, 128), jnp.int32),
            pltpu.VMEM((128, 128), jnp.float32),
            pltpu.VMEM((128, 128), jnp.float32),
            pltpu.SemaphoreType.DMA,
            pltpu.SemaphoreType.DMA,
        ],
    )
    def k(tabA_hbm, tabB_hbm, idx_hbm, out_hbm, idx_v, ra, rb, sa, sb):
        c = lax.axis_index("c")
        s = lax.axis_index("s")
        w = s * NC + c
        for t in range(NT):
            tab = tabA_hbm.at[t] if t < 3 else tabB_hbm.at[t - 3]
            pltpu.sync_copy(idx_hbm.at[t].at[w], idx_v)
            cpa = pltpu.async_copy(tab.at[idx_v.at[0]], ra, sa)

            def body(i, _):
                j0 = 2 * i
                j1 = j0 + 1
                cpb = pltpu.async_copy(tab.at[idx_v.at[j1]], rb, sb)
                cpa2 = pltpu.make_async_copy(tab.at[idx_v.at[0]], ra, sa)
                cpa2.wait()
                pltpu.sync_copy(ra, out_hbm.at[t].at[w].at[j0])
                nxt = j0 + 2

                @pl.when(nxt < G_CH)
                def _():
                    pltpu.async_copy(tab.at[idx_v.at[nxt]], ra, sa)

                cpb.wait()
                pltpu.sync_copy(rb, out_hbm.at[t].at[w].at[j1])
                return 0

            lax.fori_loop(0, G_CH // 2, body, 0)

        del cpa
        return None

    out = k(tabA, tabB, idx4)
    return out.reshape(NT, E_PAD, 128)


def _sc_seg_s(a3, dst3):
    a_sc = a3.reshape(3, NS, S_CH, 128)
    d_sc = dst3.reshape(3, NS, S_CH, 128)

    @functools.partial(
        pl.kernel,
        out_type=jax.ShapeDtypeStruct((3 * S_SZ,), jnp.float32),
        mesh=plsc.VectorSubcoreMesh(**_SC_MESH),
        scratch_types=[
            pltpu.VMEM_SHARED((S_SZ,), jnp.float32),
            pltpu.VMEM((2560,), jnp.float32),
            pltpu.VMEM((S_CH, 128), jnp.float32),
            pltpu.VMEM((S_CH, 128), jnp.int32),
        ],
    )
    def k(a_sc_hbm, d_sc_hbm, out_hbm, s_sp, zb, av, dv):
        c = lax.axis_index("c")
        s = lax.axis_index("s")

        def zero16(i, _):
            zb[pl.ds(i * 16, 16)] = jnp.zeros((16,), jnp.float32)
            return 0

        lax.fori_loop(0, 160, zero16, 0)

        for l in range(3):
            owner = 0 if l < 2 else 1

            @pl.when(c == owner)
            def _():
                pltpu.sync_copy(zb, s_sp.at[pl.ds(s * 2560, 2560)])
                plsc.subcore_barrier()
                pltpu.sync_copy(a_sc_hbm.at[l].at[s], av)
                pltpu.sync_copy(d_sc_hbm.at[l].at[s], dv)

                def scat(j, _):
                    pltpu.sync_copy(av.at[j], s_sp.at[dv.at[j]], add=True)
                    return 0

                lax.fori_loop(0, S_CH, scat, 0)
                plsc.subcore_barrier()
                pltpu.sync_copy(s_sp.at[pl.ds(s * 2560, 2560)], zb)
                pltpu.sync_copy(zb, out_hbm.at[pl.ds(l * S_SZ + s * 2560, 2560)])

                def rezero(i, _):
                    zb[pl.ds(i * 16, 16)] = jnp.zeros((16,), jnp.float32)
                    return 0

                lax.fori_loop(0, 160, rezero, 0)

    return k(a_sc, d_sc).reshape(3, S_SZ)


def _sc_scatter_rows(W, dst, out_rows=M):
    NOP = W.shape[0]
    W5 = W.reshape(NOP, NS, SC2_CH, SC2_W, 128)
    d4 = dst.reshape(NOP, NS, SC2_CH, SC2_W)

    @functools.partial(
        pl.kernel,
        out_type=jax.ShapeDtypeStruct((NOP, out_rows, 128), jnp.float32),
        mesh=plsc.VectorSubcoreMesh(**_SC_MESH),
        scratch_types=[
            pltpu.VMEM_SHARED((SP_ROWS, 128), jnp.float32),
            pltpu.VMEM((SC2_W, 128), jnp.float32),
            pltpu.VMEM((SC2_W, 128), jnp.float32),
            pltpu.VMEM((SC2_CH, SC2_W), jnp.int32),
            pltpu.SemaphoreType.DMA,
            pltpu.SemaphoreType.DMA,
        ],
    )
    def k(W_hbm, d_hbm, out_hbm, sp, wa, wb, iv, sa, sb):
        c = lax.axis_index("c")
        s = lax.axis_index("s")
        zstripe = 632
        zlo = jnp.minimum(s * zstripe, SP_ROWS - zstripe)
        dstripe = 632
        dummy = SP_DATA + s

        def zero16(kk, _):
            wa[kk // 8, pl.ds((kk % 8) * 16, 16)] = jnp.zeros((16,),
                                                              jnp.float32)
            return 0

        for o in range(NOP):
            for r in range(2):
                lo = (r * 2 + c) * SP_DATA
                sz = SP_DATA
                pltpu.sync_copy(d_hbm.at[o].at[s], iv)
                lax.fori_loop(0, 512, zero16, 0)
                for q in range(9):
                    pltpu.sync_copy(wa, sp.at[pl.ds(zlo + q * 64, 64)])
                pltpu.sync_copy(wa.at[pl.ds(0, 56)],
                                sp.at[pl.ds(zlo + 576, 56)])
                plsc.subcore_barrier()

                def adj(j, _):
                    for i in range(SC2_W // 16):
                        d16 = iv[j, pl.ds(i * 16, 16)]
                        rel = d16 - lo
                        inb = (rel >= 0) & (rel < sz)
                        iv[j, pl.ds(i * 16, 16)] = jnp.where(inb, rel, dummy)
                    return 0

                lax.fori_loop(0, SC2_CH, adj, 0)

                pltpu.async_copy(W_hbm.at[o].at[s].at[0], wa, sa)

                def body(i, _):
                    j0 = 2 * i
                    j1 = j0 + 1
                    cpb = pltpu.async_copy(W_hbm.at[o].at[s].at[j1], wb, sb)
                    pltpu.make_async_copy(
                        W_hbm.at[o].at[s].at[0], wa, sa).wait()
                    pltpu.sync_copy(wa, sp.at[iv.at[j0]], add=True)
                    nxt = j0 + 2

                    @pl.when(nxt < SC2_CH)
                    def _():
                        pltpu.async_copy(W_hbm.at[o].at[s].at[nxt], wa, sa)

                    cpb.wait()
                    pltpu.sync_copy(wb, sp.at[iv.at[j1]], add=True)
                    return 0

                lax.fori_loop(0, SC2_CH // 2, body, 0)
                plsc.subcore_barrier()

                mylo = jnp.minimum(s * dstripe, sz - dstripe)
                for q in range(9):
                    pltpu.sync_copy(sp.at[pl.ds(mylo + q * 64, 64)], wa)
                    pltpu.sync_copy(
                        wa, out_hbm.at[o].at[pl.ds(lo + mylo + q * 64, 64)])
                pltpu.sync_copy(sp.at[pl.ds(mylo + 576, 56)],
                                wa.at[pl.ds(0, 56)])
                pltpu.sync_copy(wa.at[pl.ds(0, 56)],
                                out_hbm.at[o].at[pl.ds(lo + mylo + 576, 56)])
                plsc.subcore_barrier()

    return k(W5, d4)


_ER = 3 * E_PAD
_RB = 4096


def _logits_body(g1_ref, g2_ref, att_ref, out_ref):
    e = g1_ref[...] + g2_ref[...]
    e = jnp.where(e > 0, e, 0.2 * e)
    out_ref[...] = jnp.exp(
        jnp.sum(e * att_ref[...][None, :], axis=1, keepdims=True))


def _edge_logits(G1f, G2f, att):
    grid = (_ER // _RB,)
    out = pl.pallas_call(
        _logits_body,
        grid=grid,
        in_specs=[
            pl.BlockSpec((_RB, D), lambda i: (i, 0)),
            pl.BlockSpec((_RB, D), lambda i: (i, 0)),
            pl.BlockSpec((D,), lambda i: (0,)),
        ],
        out_specs=pl.BlockSpec((_RB, 1), lambda i: (i, 0)),
        out_shape=jax.ShapeDtypeStruct((_ER, 1), jnp.float32),
    )(G1f, G2f, att)
    return out.reshape(3, E_PAD)


def _scale_body(sc_ref, rows_ref, out_ref):
    out_ref[...] = rows_ref[...] * sc_ref[...]


def _ndiv_body(sc_ref, rows_ref, bgat_ref, out_ref):
    out_ref[...] = (rows_ref[...] / (sc_ref[...] + 1e-16)
                    + bgat_ref[...][None, :])


def _node_div(s3, raw, bgat):
    NR = 3 * S_SZ
    grid = (NR // _RB,)
    out = pl.pallas_call(
        _ndiv_body,
        grid=grid,
        in_specs=[
            pl.BlockSpec((_RB, 1), lambda i: (i, 0)),
            pl.BlockSpec((_RB, D), lambda i: (i, 0)),
            pl.BlockSpec((D,), lambda i: (0,)),
        ],
        out_specs=pl.BlockSpec((_RB, D), lambda i: (i, 0)),
        out_shape=jax.ShapeDtypeStruct((NR, D), jnp.float32),
    )(s3.reshape(NR, 1), raw.reshape(NR, D), bgat)
    return out.reshape(3, S_SZ, D)


def _edge_scale(scale3, rows):
    grid = (_ER // _RB,)
    out = pl.pallas_call(
        _scale_body,
        grid=grid,
        in_specs=[
            pl.BlockSpec((_RB, 1), lambda i: (i, 0)),
            pl.BlockSpec((_RB, D), lambda i: (i, 0)),
        ],
        out_specs=pl.BlockSpec((_RB, D), lambda i: (i, 0)),
        out_shape=jax.ShapeDtypeStruct((_ER, D), jnp.float32),
    )(scale3.reshape(_ER, 1), rows.reshape(_ER, D))
    return out.reshape(3, E_PAD, D)


def _gt_body(aggd_ref, aggs_ref, z_ref, Wg_ref, bg_ref, out_ref):
    m = jnp.mean(aggd_ref[...] + aggs_ref[...], axis=2)
    xm = jnp.mean(z_ref[...], axis=2)
    P = m.shape[1]
    g = jnp.dot(m.reshape(L * P, D), Wg_ref[...],
                preferred_element_type=jnp.float32) + bg_ref[...]
    out_ref[...] = g.reshape(L, P, D) + xm


def _gt_kernel(aggd, aggs, z14, Wg, bg):
    P = 1000
    grid = (N_PED // P,)
    return pl.pallas_call(
        _gt_body,
        grid=grid,
        in_specs=[
            pl.BlockSpec((L, P, PATCH, D), lambda i: (0, i, 0, 0)),
            pl.BlockSpec((L, P, PATCH, D), lambda i: (0, i, 0, 0)),
            pl.BlockSpec((L, P, PATCH, D), lambda i: (0, i, 0, 0)),
            pl.BlockSpec((D, D), lambda i: (0, 0)),
            pl.BlockSpec((D,), lambda i: (0,)),
        ],
        out_specs=pl.BlockSpec((L, P, D), lambda i: (0, i, 0)),
        out_shape=jax.ShapeDtypeStruct((L, N_PED, D), jnp.float32),
    )(aggd, aggs, z14, Wg, bg)


def kernel(ped_obs, vel, nodes, edge_index, edge_attr, edge_edge_index, Wn, bn,
           Wa, ba, Wv, bv, We, be, Wh, bh, Wc, bc, Wgl, Wgr, att, bgat, Wg, bg,
           step_placeholder, Wout, bout, gF, bF,
           enc0_Wq, enc0_bq, enc0_Wk, enc0_bk, enc0_Wv, enc0_bv, enc0_Wo,
           enc0_bo, enc0_g1, enc0_b1, enc0_W1, enc0_bb1, enc0_W2, enc0_bb2,
           enc0_g2, enc0_b2,
           enc1_Wq, enc1_bq, enc1_Wk, enc1_bk, enc1_Wv, enc1_bv, enc1_Wo,
           enc1_bo, enc1_g1, enc1_b1, enc1_W1, enc1_bb1, enc1_W2, enc1_bb2,
           enc1_g2, enc1_b2,
           W_pos):
    vx = vel[:, :, 0]
    vy = vel[:, :, 1]
    vpx = jnp.concatenate([vx[:, :1], vx[:, :-1]], axis=1)
    vpy = jnp.concatenate([vy[:, :1], vy[:, :-1]], axis=1)
    nrm = jnp.sqrt(vx * vx + vy * vy)
    nrm_p = jnp.sqrt(vpx * vpx + vpy * vpy)
    cos = (vpx * vx + vpy * vy) / ((nrm_p + EPS) * (nrm + EPS))
    ang = jnp.arccos(jnp.clip(cos, -1.0, 1.0))
    nf = _featurize(nodes, ang, vx, vy, Wn, bn, Wa, ba, Wv, bv)

    z1 = jnp.stack([nf[:, i * STRIDE:i * STRIDE + PATCH] for i in range(L)],
                   axis=0).reshape(L, M, D)
    hl_all, hr_all = _proj(z1.reshape(L * M, D), Wgl, Wgr)
    hl_all = hl_all.reshape(L, M, D)
    hr_all = hr_all.reshape(L, M, D)

    c = (We @ Wh @ Wc)[0]
    dvec = (be @ Wh + bh) @ Wc + bc

    pad_e = E_PAD - E
    src = edge_index[:, 0, :]
    dst = edge_index[:, 1, :]
    zpad = jnp.zeros((L, pad_e), jnp.int32)
    mpad = jnp.full((L, pad_e), M, jnp.int32)
    srcg = jnp.concatenate([src, zpad], axis=1)
    dstg = jnp.concatenate([dst, zpad], axis=1)
    srcs = jnp.concatenate([src, mpad], axis=1)
    dsts = jnp.concatenate([dst, mpad], axis=1)

    idx6 = jnp.concatenate([srcg, dstg], axis=0)
    G = _sc_gather(hl_all, hr_all, idx6)
    G1f = G[:3].reshape(_ER, D)
    G2f = G[3:].reshape(_ER, D)

    a3 = _edge_logits(G1f, G2f, att)

    s3 = _sc_seg_s(a3, dsts)

    Wrows = _edge_scale(a3, G[:3])
    ne_raw = _sc_scatter_rows(Wrows, dsts, out_rows=S_SZ)
    ne3 = _node_div(s3, ne_raw, bgat)

    G2nd = _sc_gather(ne3, ne3, idx6)

    ea_c = jnp.where(jnp.isnan(edge_attr) | jnp.isinf(edge_attr), 0.0,
                     edge_attr)
    ea_p = jnp.concatenate([ea_c, jnp.zeros((L, pad_e), jnp.float32)], axis=1)
    en0 = ea_p * c[0] + dvec[0]
    en1 = ea_p * c[1] + dvec[1]
    W1 = _edge_scale(en0, G2nd[:3])
    W2 = _edge_scale(en1, G2nd[3:])

    aggd = _sc_scatter_rows(W1, dsts)
    aggs_ = _sc_scatter_rows(W2, srcs)

    gt3 = _gt_kernel(aggd.reshape(L, N_PED, PATCH, D),
                     aggs_.reshape(L, N_PED, PATCH, D),
                     z1.reshape(L, N_PED, PATCH, D), Wg, bg)
    gt = jnp.transpose(gt3, (1, 0, 2))

    sp = step_placeholder[0]
    enc_params = [enc0_Wq, enc0_bq, enc0_Wk, enc0_bk, enc0_Wv, enc0_bv,
                  enc0_Wo, enc0_bo, enc0_g1, enc0_b1, enc0_W1, enc0_bb1,
                  enc0_W2, enc0_bb2, enc0_g2, enc0_b2,
                  enc1_Wq, enc1_bq, enc1_Wk, enc1_bk, enc1_Wv, enc1_bv,
                  enc1_Wo, enc1_bo, enc1_g1, enc1_b1, enc1_W1, enc1_bb1,
                  enc1_W2, enc1_bb2, enc1_g2, enc1_b2]
    return _encoder(gt, sp, W_pos, enc_params, gF, bF, Wout, bout)


# speedup vs baseline: 1.5624x; 1.0043x over previous
import functools
import math

import jax
import jax.numpy as jnp
from jax import lax
from jax.experimental import pallas as pl
from jax.experimental.pallas import tpu as pltpu
from jax.experimental.pallas import tpu_sc as plsc

N_PED = 10000
T = 8
PATCH = 4
STRIDE = 2
L = 3
D = 128
PRED = 12
E = 160000
M = N_PED * PATCH
H = 4
DH = D // H
S = L + PRED
EPS = 1e-6


def _feat_body(nodes_ref, ang_ref, vx_ref, vy_ref, Wn_ref, bn_ref, Wa_ref,
               ba_ref, Wv_ref, bv_ref, out_ref):
    nodes = nodes_ref[...]
    ang = ang_ref[...]
    vx = vx_ref[...]
    vy = vy_ref[...]
    f0 = nodes[:, :, None] * Wn_ref[0][None, None, :] + bn_ref[...][None, None, :]
    f1 = ang[:, :, None] * Wa_ref[0][None, None, :] + ba_ref[...][None, None, :]
    f2 = (vx[:, :, None] * Wv_ref[0][None, None, :]
          + vy[:, :, None] * Wv_ref[1][None, None, :]
          + bv_ref[...][None, None, :])
    out_ref[...] = jnp.concatenate([f0, f1, f2], axis=-1)


def _featurize(nodes, ang, vx, vy, Wn, bn, Wa, ba, Wv, bv):
    P = 2000
    grid = (N_PED // P,)
    return pl.pallas_call(
        _feat_body,
        grid=grid,
        in_specs=[
            pl.BlockSpec((P, T), lambda i: (i, 0)),
            pl.BlockSpec((P, T), lambda i: (i, 0)),
            pl.BlockSpec((P, T), lambda i: (i, 0)),
            pl.BlockSpec((P, T), lambda i: (i, 0)),
            pl.BlockSpec((1, 32), lambda i: (0, 0)),
            pl.BlockSpec((32,), lambda i: (0,)),
            pl.BlockSpec((1, 32), lambda i: (0, 0)),
            pl.BlockSpec((32,), lambda i: (0,)),
            pl.BlockSpec((2, 64), lambda i: (0, 0)),
            pl.BlockSpec((64,), lambda i: (0,)),
        ],
        out_specs=pl.BlockSpec((P, T, D), lambda i: (i, 0, 0)),
        out_shape=jax.ShapeDtypeStruct((N_PED, T, D), jnp.float32),
    )(nodes, ang, vx, vy, Wn, bn, Wa, ba, Wv, bv)


def _proj_body(x_ref, Wgl_ref, Wgr_ref, hl_ref, hr_ref):
    x = x_ref[...]
    hl_ref[...] = jnp.dot(x, Wgl_ref[...], preferred_element_type=jnp.float32)
    hr_ref[...] = jnp.dot(x, Wgr_ref[...], preferred_element_type=jnp.float32)


def _proj(z1_flat, Wgl, Wgr):
    R = z1_flat.shape[0]
    P = 6000
    grid = (R // P,)
    return pl.pallas_call(
        _proj_body,
        grid=grid,
        in_specs=[
            pl.BlockSpec((P, D), lambda i: (i, 0)),
            pl.BlockSpec((D, D), lambda i: (0, 0)),
            pl.BlockSpec((D, D), lambda i: (0, 0)),
        ],
        out_specs=[
            pl.BlockSpec((P, D), lambda i: (i, 0)),
            pl.BlockSpec((P, D), lambda i: (i, 0)),
        ],
        out_shape=[
            jax.ShapeDtypeStruct((R, D), jnp.float32),
            jax.ShapeDtypeStruct((R, D), jnp.float32),
        ],
    )(z1_flat, Wgl, Wgr)


def _ln_rows(x, g, b):
    m = jnp.mean(x, axis=-1, keepdims=True)
    v = jnp.mean((x - m) ** 2, axis=-1, keepdims=True)
    return (x - m) * jax.lax.rsqrt(v + 1e-5) * g + b


def _enc_layer(xx, P, pr):
    q = jnp.dot(xx, pr['Wq'], preferred_element_type=jnp.float32) + pr['bq']
    k = jnp.dot(xx, pr['Wk'], preferred_element_type=jnp.float32) + pr['bk']
    v = jnp.dot(xx, pr['Wv'], preferred_element_type=jnp.float32) + pr['bv']
    q3 = q.reshape(P, S, D)
    k3 = k.reshape(P, S, D)
    v3 = v.reshape(P, S, D)
    scale = 1.0 / math.sqrt(DH)
    o_heads = []
    for h in range(H):
        qh = q3[:, :, h * DH:(h + 1) * DH]
        kh = k3[:, :, h * DH:(h + 1) * DH]
        vh = v3[:, :, h * DH:(h + 1) * DH]
        sc = jax.lax.dot_general(
            qh, kh, (((2,), (2,)), ((0,), (0,))),
            preferred_element_type=jnp.float32) * scale
        mx = jnp.max(sc, axis=-1, keepdims=True)
        ex = jnp.exp(sc - mx)
        aw = ex / jnp.sum(ex, axis=-1, keepdims=True)
        oh = jax.lax.dot_general(
            aw, vh, (((2,), (1,)), ((0,), (0,))),
            preferred_element_type=jnp.float32)
        o_heads.append(oh)
    o = jnp.concatenate(o_heads, axis=-1).reshape(P * S, D)
    o = jnp.dot(o, pr['Wo'], preferred_element_type=jnp.float32) + pr['bo']
    xx = xx + o
    xx = _ln_rows(xx, pr['g1'], pr['b1'])
    y = jnp.dot(xx, pr['W1'], preferred_element_type=jnp.float32) + pr['bb1']
    y = jax.nn.gelu(y)
    y = jnp.dot(y, pr['W2'], preferred_element_type=jnp.float32) + pr['bb2']
    xx = _ln_rows(xx + y, pr['g2'], pr['b2'])
    return xx


def _enc_body(gt_ref, sp_ref, wpos_ref, *rest):
    names = ['Wq', 'bq', 'Wk', 'bk', 'Wv', 'bv', 'Wo', 'bo',
             'g1', 'b1', 'W1', 'bb1', 'W2', 'bb2', 'g2', 'b2']
    p0 = {n: rest[i][...] for i, n in enumerate(names)}
    p1 = {n: rest[16 + i][...] for i, n in enumerate(names)}
    gF = rest[32][...]
    bF = rest[33][...]
    Wout = rest[34][...]
    bout = rest[35][...]
    out_ref = rest[36]

    gt = gt_ref[...]
    sp = sp_ref[...]
    P = gt.shape[0]
    ph = jnp.broadcast_to(sp[:, None, :], (P, PRED, D))
    xx = jnp.concatenate([gt, ph], axis=1) + wpos_ref[...][None, :, :]
    xx = xx.reshape(P * S, D)
    xx = _enc_layer(xx, P, p0)
    xx = _enc_layer(xx, P, p1)
    xx = _ln_rows(xx, gF, bF)
    out = jnp.dot(xx, Wout, preferred_element_type=jnp.float32) + bout
    out_ref[...] = out.reshape(P, S, 5)


def _encoder(gt, sp, W_pos, enc_params, gF, bF, Wout, bout):
    P = 400
    grid = (N_PED // P,)
    in_specs = [
        pl.BlockSpec((P, L, D), lambda i: (i, 0, 0)),
        pl.BlockSpec((P, D), lambda i: (i, 0)),
        pl.BlockSpec((S, D), lambda i: (0, 0)),
    ]
    args = [gt, sp, W_pos]
    for arr in enc_params:
        in_specs.append(pl.BlockSpec(arr.shape, lambda i, nd=arr.ndim: (0,) * nd))
        args.append(arr)
    for arr in (gF, bF, Wout, bout):
        in_specs.append(pl.BlockSpec(arr.shape, lambda i, nd=arr.ndim: (0,) * nd))
        args.append(arr)
    return pl.pallas_call(
        _enc_body,
        grid=grid,
        in_specs=in_specs,
        out_specs=pl.BlockSpec((P, S, 5), lambda i: (i, 0, 0)),
        out_shape=jax.ShapeDtypeStruct((N_PED, S, 5), jnp.float32),
    )(*args)


E_PAD = 163840
NC = 2
NS = 16
NW = NC * NS
G_CH = 40
S_CH = 80
S_SZ = 40960
SP_DATA = 10000
SP_ROWS = SP_DATA + NS
SC2_CH = 160
SC2_W = 64
_SC_MESH = dict(core_axis_name="c", subcore_axis_name="s")


def _sc_gather(tabA, tabB, idx):
    NT = 6
    NSUP = G_CH // 2
    idx4 = idx.reshape(NT, NW, G_CH, 128)

    @functools.partial(
        pl.kernel,
        out_type=jax.ShapeDtypeStruct((NT, NW, NSUP, 256, 128), jnp.float32),
        mesh=plsc.VectorSubcoreMesh(**_SC_MESH),
        scratch_types=[
            pltpu.VMEM((G_CH, 128), jnp.int32),
            pltpu.VMEM((256, 128), jnp.float32),
            pltpu.VMEM((256, 128), jnp.float32),
            pltpu.SemaphoreType.DMA,
            pltpu.SemaphoreType.DMA,
        ],
    )
    def k(tabA_hbm, tabB_hbm, idx_hbm, out_hbm, idx_v, ra, rb, sa, sb):
        c = lax.axis_index("c")
        s = lax.axis_index("s")
        w = s * NC + c

        def fire2(tab, buf, sem, sup):
            pltpu.async_copy(tab.at[idx_v.at[2 * sup]],
                             buf.at[pl.ds(0, 128)], sem)
            pltpu.async_copy(tab.at[idx_v.at[2 * sup + 1]],
                             buf.at[pl.ds(128, 128)], sem)

        def drain2(tab, buf, sem):
            pltpu.make_async_copy(tab.at[idx_v.at[0]],
                                  buf.at[pl.ds(0, 128)], sem).wait()
            pltpu.make_async_copy(tab.at[idx_v.at[0]],
                                  buf.at[pl.ds(128, 128)], sem).wait()

        for t in range(NT):
            tab = tabA_hbm.at[t] if t < 3 else tabB_hbm.at[t - 3]
            pltpu.sync_copy(idx_hbm.at[t].at[w], idx_v)
            fire2(tab, ra, sa, 0)

            def body(i, _):
                u0 = 2 * i
                u1 = u0 + 1
                fire2(tab, rb, sb, u1)
                drain2(tab, ra, sa)
                pltpu.sync_copy(ra, out_hbm.at[t].at[w].at[u0])
                nxt = u0 + 2

                @pl.when(nxt < NSUP)
                def _():
                    fire2(tab, ra, sa, nxt)

                drain2(tab, rb, sb)
                pltpu.sync_copy(rb, out_hbm.at[t].at[w].at[u1])
                return 0

            lax.fori_loop(0, NSUP // 2, body, 0)

    out = k(tabA, tabB, idx4)
    return out.reshape(NT, E_PAD, 128)


def _sc_seg_s(a3, dst3):
    a_sc = a3.reshape(3, NS, S_CH, 128)
    d_sc = dst3.reshape(3, NS, S_CH, 128)

    @functools.partial(
        pl.kernel,
        out_type=jax.ShapeDtypeStruct((3 * S_SZ,), jnp.float32),
        mesh=plsc.VectorSubcoreMesh(**_SC_MESH),
        scratch_types=[
            pltpu.VMEM_SHARED((S_SZ,), jnp.float32),
            pltpu.VMEM((2560,), jnp.float32),
            pltpu.VMEM((S_CH, 128), jnp.float32),
            pltpu.VMEM((S_CH, 128), jnp.int32),
        ],
    )
    def k(a_sc_hbm, d_sc_hbm, out_hbm, s_sp, zb, av, dv):
        c = lax.axis_index("c")
        s = lax.axis_index("s")

        def zero16(i, _):
            zb[pl.ds(i * 16, 16)] = jnp.zeros((16,), jnp.float32)
            return 0

        lax.fori_loop(0, 160, zero16, 0)

        for l in range(3):
            owner = 0 if l < 2 else 1

            @pl.when(c == owner)
            def _():
                pltpu.sync_copy(zb, s_sp.at[pl.ds(s * 2560, 2560)])
                plsc.subcore_barrier()
                pltpu.sync_copy(a_sc_hbm.at[l].at[s], av)
                pltpu.sync_copy(d_sc_hbm.at[l].at[s], dv)

                def scat(j, _):
                    pltpu.sync_copy(av.at[j], s_sp.at[dv.at[j]], add=True)
                    return 0

                lax.fori_loop(0, S_CH, scat, 0)
                plsc.subcore_barrier()
                pltpu.sync_copy(s_sp.at[pl.ds(s * 2560, 2560)], zb)
                pltpu.sync_copy(zb, out_hbm.at[pl.ds(l * S_SZ + s * 2560, 2560)])

                def rezero(i, _):
                    zb[pl.ds(i * 16, 16)] = jnp.zeros((16,), jnp.float32)
                    return 0

                lax.fori_loop(0, 160, rezero, 0)

    return k(a_sc, d_sc).reshape(3, S_SZ)


def _sc_scatter_rows(W, dst, out_rows=M):
    NOP = W.shape[0]
    W5 = W.reshape(NOP, NS, SC2_CH, SC2_W, 128)
    d4 = dst.reshape(NOP, NS, SC2_CH, SC2_W)

    @functools.partial(
        pl.kernel,
        out_type=jax.ShapeDtypeStruct((NOP, out_rows, 128), jnp.float32),
        mesh=plsc.VectorSubcoreMesh(**_SC_MESH),
        scratch_types=[
            pltpu.VMEM_SHARED((SP_ROWS, 128), jnp.float32),
            pltpu.VMEM((SC2_W, 128), jnp.float32),
            pltpu.VMEM((SC2_W, 128), jnp.float32),
            pltpu.VMEM((SC2_CH, SC2_W), jnp.int32),
            pltpu.SemaphoreType.DMA,
            pltpu.SemaphoreType.DMA,
        ],
    )
    def k(W_hbm, d_hbm, out_hbm, sp, wa, wb, iv, sa, sb):
        c = lax.axis_index("c")
        s = lax.axis_index("s")
        zstripe = 632
        zlo = jnp.minimum(s * zstripe, SP_ROWS - zstripe)
        dstripe = 632
        dummy = SP_DATA + s

        def zero16(kk, _):
            wa[kk // 8, pl.ds((kk % 8) * 16, 16)] = jnp.zeros((16,),
                                                              jnp.float32)
            return 0

        for o in range(NOP):
            for r in range(2):
                lo = (r * 2 + c) * SP_DATA
                sz = SP_DATA
                pltpu.sync_copy(d_hbm.at[o].at[s], iv)
                lax.fori_loop(0, 512, zero16, 0)
                for q in range(9):
                    pltpu.sync_copy(wa, sp.at[pl.ds(zlo + q * 64, 64)])
                pltpu.sync_copy(wa.at[pl.ds(0, 56)],
                                sp.at[pl.ds(zlo + 576, 56)])
                plsc.subcore_barrier()

                def adj(j, _):
                    for i in range(SC2_W // 16):
                        d16 = iv[j, pl.ds(i * 16, 16)]
                        rel = d16 - lo
                        inb = (rel >= 0) & (rel < sz)
                        iv[j, pl.ds(i * 16, 16)] = jnp.where(inb, rel, dummy)
                    return 0

                lax.fori_loop(0, SC2_CH, adj, 0)

                pltpu.async_copy(W_hbm.at[o].at[s].at[0], wa, sa)

                def body(i, _):
                    j0 = 2 * i
                    j1 = j0 + 1
                    cpb = pltpu.async_copy(W_hbm.at[o].at[s].at[j1], wb, sb)
                    pltpu.make_async_copy(
                        W_hbm.at[o].at[s].at[0], wa, sa).wait()
                    pltpu.sync_copy(wa, sp.at[iv.at[j0]], add=True)
                    nxt = j0 + 2

                    @pl.when(nxt < SC2_CH)
                    def _():
                        pltpu.async_copy(W_hbm.at[o].at[s].at[nxt], wa, sa)

                    cpb.wait()
                    pltpu.sync_copy(wb, sp.at[iv.at[j1]], add=True)
                    return 0

                lax.fori_loop(0, SC2_CH // 2, body, 0)
                plsc.subcore_barrier()

                mylo = jnp.minimum(s * dstripe, sz - dstripe)
                for q in range(9):
                    pltpu.sync_copy(sp.at[pl.ds(mylo + q * 64, 64)], wa)
                    pltpu.sync_copy(
                        wa, out_hbm.at[o].at[pl.ds(lo + mylo + q * 64, 64)])
                pltpu.sync_copy(sp.at[pl.ds(mylo + 576, 56)],
                                wa.at[pl.ds(0, 56)])
                pltpu.sync_copy(wa.at[pl.ds(0, 56)],
                                out_hbm.at[o].at[pl.ds(lo + mylo + 576, 56)])
                plsc.subcore_barrier()

    return k(W5, d4)


_ER = 3 * E_PAD
_RB = 4096


def _logits_body(g1_ref, g2_ref, att_ref, out_ref):
    e = g1_ref[...] + g2_ref[...]
    e = jnp.where(e > 0, e, 0.2 * e)
    out_ref[...] = jnp.exp(
        jnp.sum(e * att_ref[...][None, :], axis=1, keepdims=True))


def _edge_logits(G1f, G2f, att):
    grid = (_ER // _RB,)
    out = pl.pallas_call(
        _logits_body,
        grid=grid,
        in_specs=[
            pl.BlockSpec((_RB, D), lambda i: (i, 0)),
            pl.BlockSpec((_RB, D), lambda i: (i, 0)),
            pl.BlockSpec((D,), lambda i: (0,)),
        ],
        out_specs=pl.BlockSpec((_RB, 1), lambda i: (i, 0)),
        out_shape=jax.ShapeDtypeStruct((_ER, 1), jnp.float32),
    )(G1f, G2f, att)
    return out.reshape(3, E_PAD)


def _scale_body(sc_ref, rows_ref, out_ref):
    out_ref[...] = rows_ref[...] * sc_ref[...]


def _ndiv_body(sc_ref, rows_ref, bgat_ref, out_ref):
    out_ref[...] = (rows_ref[...] / (sc_ref[...] + 1e-16)
                    + bgat_ref[...][None, :])


def _node_div(s3, raw, bgat):
    NR = 3 * S_SZ
    grid = (NR // _RB,)
    out = pl.pallas_call(
        _ndiv_body,
        grid=grid,
        in_specs=[
            pl.BlockSpec((_RB, 1), lambda i: (i, 0)),
            pl.BlockSpec((_RB, D), lambda i: (i, 0)),
            pl.BlockSpec((D,), lambda i: (0,)),
        ],
        out_specs=pl.BlockSpec((_RB, D), lambda i: (i, 0)),
        out_shape=jax.ShapeDtypeStruct((NR, D), jnp.float32),
    )(s3.reshape(NR, 1), raw.reshape(NR, D), bgat)
    return out.reshape(3, S_SZ, D)


def _edge_scale(scale3, rows):
    grid = (_ER // _RB,)
    out = pl.pallas_call(
        _scale_body,
        grid=grid,
        in_specs=[
            pl.BlockSpec((_RB, 1), lambda i: (i, 0)),
            pl.BlockSpec((_RB, D), lambda i: (i, 0)),
        ],
        out_specs=pl.BlockSpec((_RB, D), lambda i: (i, 0)),
        out_shape=jax.ShapeDtypeStruct((_ER, D), jnp.float32),
    )(scale3.reshape(_ER, 1), rows.reshape(_ER, D))
    return out.reshape(3, E_PAD, D)


def _gt_body(aggd_ref, aggs_ref, z_ref, Wg_ref, bg_ref, out_ref):
    m = jnp.mean(aggd_ref[...] + aggs_ref[...], axis=2)
    xm = jnp.mean(z_ref[...], axis=2)
    P = m.shape[1]
    g = jnp.dot(m.reshape(L * P, D), Wg_ref[...],
                preferred_element_type=jnp.float32) + bg_ref[...]
    out_ref[...] = g.reshape(L, P, D) + xm


def _gt_kernel(aggd, aggs, z14, Wg, bg):
    P = 1000
    grid = (N_PED // P,)
    return pl.pallas_call(
        _gt_body,
        grid=grid,
        in_specs=[
            pl.BlockSpec((L, P, PATCH, D), lambda i: (0, i, 0, 0)),
            pl.BlockSpec((L, P, PATCH, D), lambda i: (0, i, 0, 0)),
            pl.BlockSpec((L, P, PATCH, D), lambda i: (0, i, 0, 0)),
            pl.BlockSpec((D, D), lambda i: (0, 0)),
            pl.BlockSpec((D,), lambda i: (0,)),
        ],
        out_specs=pl.BlockSpec((L, P, D), lambda i: (0, i, 0)),
        out_shape=jax.ShapeDtypeStruct((L, N_PED, D), jnp.float32),
    )(aggd, aggs, z14, Wg, bg)


def kernel(ped_obs, vel, nodes, edge_index, edge_attr, edge_edge_index, Wn, bn,
           Wa, ba, Wv, bv, We, be, Wh, bh, Wc, bc, Wgl, Wgr, att, bgat, Wg, bg,
           step_placeholder, Wout, bout, gF, bF,
           enc0_Wq, enc0_bq, enc0_Wk, enc0_bk, enc0_Wv, enc0_bv, enc0_Wo,
           enc0_bo, enc0_g1, enc0_b1, enc0_W1, enc0_bb1, enc0_W2, enc0_bb2,
           enc0_g2, enc0_b2,
           enc1_Wq, enc1_bq, enc1_Wk, enc1_bk, enc1_Wv, enc1_bv, enc1_Wo,
           enc1_bo, enc1_g1, enc1_b1, enc1_W1, enc1_bb1, enc1_W2, enc1_bb2,
           enc1_g2, enc1_b2,
           W_pos):
    vx = vel[:, :, 0]
    vy = vel[:, :, 1]
    vpx = jnp.concatenate([vx[:, :1], vx[:, :-1]], axis=1)
    vpy = jnp.concatenate([vy[:, :1], vy[:, :-1]], axis=1)
    nrm = jnp.sqrt(vx * vx + vy * vy)
    nrm_p = jnp.sqrt(vpx * vpx + vpy * vpy)
    cos = (vpx * vx + vpy * vy) / ((nrm_p + EPS) * (nrm + EPS))
    ang = jnp.arccos(jnp.clip(cos, -1.0, 1.0))
    nf = _featurize(nodes, ang, vx, vy, Wn, bn, Wa, ba, Wv, bv)

    z1 = jnp.stack([nf[:, i * STRIDE:i * STRIDE + PATCH] for i in range(L)],
                   axis=0).reshape(L, M, D)
    hl_all, hr_all = _proj(z1.reshape(L * M, D), Wgl, Wgr)
    hl_all = hl_all.reshape(L, M, D)
    hr_all = hr_all.reshape(L, M, D)

    c = (We @ Wh @ Wc)[0]
    dvec = (be @ Wh + bh) @ Wc + bc

    pad_e = E_PAD - E
    src = edge_index[:, 0, :]
    dst = edge_index[:, 1, :]
    zpad = jnp.zeros((L, pad_e), jnp.int32)
    mpad = jnp.full((L, pad_e), M, jnp.int32)
    srcg = jnp.concatenate([src, zpad], axis=1)
    dstg = jnp.concatenate([dst, zpad], axis=1)
    srcs = jnp.concatenate([src, mpad], axis=1)
    dsts = jnp.concatenate([dst, mpad], axis=1)

    idx6 = jnp.concatenate([srcg, dstg], axis=0)
    G = _sc_gather(hl_all, hr_all, idx6)
    G1f = G[:3].reshape(_ER, D)
    G2f = G[3:].reshape(_ER, D)

    a3 = _edge_logits(G1f, G2f, att)

    s3 = _sc_seg_s(a3, dsts)

    Wrows = _edge_scale(a3, G[:3])
    ne_raw = _sc_scatter_rows(Wrows, dsts, out_rows=S_SZ)
    ne3 = _node_div(s3, ne_raw, bgat)

    G2nd = _sc_gather(ne3, ne3, idx6)

    ea_c = jnp.where(jnp.isnan(edge_attr) | jnp.isinf(edge_attr), 0.0,
                     edge_attr)
    ea_p = jnp.concatenate([ea_c, jnp.zeros((L, pad_e), jnp.float32)], axis=1)
    en0 = ea_p * c[0] + dvec[0]
    en1 = ea_p * c[1] + dvec[1]
    W1 = _edge_scale(en0, G2nd[:3])
    W2 = _edge_scale(en1, G2nd[3:])

    aggd = _sc_scatter_rows(W1, dsts)
    aggs_ = _sc_scatter_rows(W2, srcs)

    gt3 = _gt_kernel(aggd.reshape(L, N_PED, PATCH, D),
                     aggs_.reshape(L, N_PED, PATCH, D),
                     z1.reshape(L, N_PED, PATCH, D), Wg, bg)
    gt = jnp.transpose(gt3, (1, 0, 2))

    sp = step_placeholder[0]
    enc_params = [enc0_Wq, enc0_bq, enc0_Wk, enc0_bk, enc0_Wv, enc0_bv,
                  enc0_Wo, enc0_bo, enc0_g1, enc0_b1, enc0_W1, enc0_bb1,
                  enc0_W2, enc0_bb2, enc0_g2, enc0_b2,
                  enc1_Wq, enc1_bq, enc1_Wk, enc1_bk, enc1_Wv, enc1_bv,
                  enc1_Wo, enc1_bo, enc1_g1, enc1_b1, enc1_W1, enc1_bb1,
                  enc1_W2, enc1_bb2, enc1_g2, enc1_b2]
    return _encoder(gt, sp, W_pos, enc_params, gF, bF, Wout, bout)
